# baseline (speedup 1.0000x reference)
import functools

import jax
import jax.numpy as jnp
from jax import lax
from jax.experimental import pallas as pl
from jax.experimental.pallas import tpu as pltpu
from jax.experimental.pallas import tpu_sc as plsc

IN_DIM = 128
NUM_HEADS = 8
OUT_DIM = 16
HID = NUM_HEADS * OUT_DIM
N_NODES = 10000
N_EDGES = 320000
CLAMP = 5.0

NC = 2
NS = 16
NW = NC * NS
EPW = N_EDGES // NW
CHUNK = 40
NCHUNK = EPW // CHUNK
ACC_W = 144
N_ACC = 10240
ROWS_PER_TILE = N_ACC // NS


def _interleave_perm():
    p = []
    for j in range(NUM_HEADS // 2):
        for d in range(OUT_DIM):
            p.append(32 * j + d)
            p.append(32 * j + OUT_DIM + d)
    return jnp.array(p, dtype=jnp.int32)


def _proj_nodes_body(x_ref, wq_ref, bq_ref, wk_ref, wv_ref, q_ref, kv_ref):
    xb = x_ref[...]
    q = jnp.dot(xb, wq_ref[...], preferred_element_type=jnp.float32) + bq_ref[...]
    q_ref[...] = q.astype(jnp.bfloat16)
    kv_ref[:, :HID] = jnp.dot(
        xb, wk_ref[...], preferred_element_type=jnp.float32).astype(jnp.bfloat16)
    kv_ref[:, HID:] = jnp.dot(
        xb, wv_ref[...], preferred_element_type=jnp.float32).astype(jnp.bfloat16)


def _proj_edges_body(ea_ref, we1_ref, be1_ref, e_ref):
    e_ref[...] = (
        jnp.dot(ea_ref[...], we1_ref[...], preferred_element_type=jnp.float32)
        + be1_ref[...]
    ).astype(jnp.bfloat16)


def _finalize_body(part_ref, out_ref):
    s = part_ref[0] + part_ref[1]
    pieces = []
    for h in range(NUM_HEADS):
        num = s[:, h * OUT_DIM:(h + 1) * OUT_DIM]
        den = s[:, HID + h:HID + h + 1] + 1e-16
        pieces.append(num / den)
    out_ref[...] = jnp.concatenate(pieces, axis=1)


def _lane_perm(x, idx):
    dn = lax.GatherDimensionNumbers(
        offset_dims=(), collapsed_slice_dims=(0,), start_index_map=(0,))
    return lax.gather(x, idx[:, None], dn, slice_sizes=(1,),
                      mode=lax.GatherScatterMode.PROMISE_IN_BOUNDS)


def _sc_edge_body(kvt, qt, e_hbm, ei_hbm,
                  we_hbm, part_hbm,
                  srcv0, srcv1, dstv0, dstv1, kvf0, kvf1, qf0, qf1,
                  ef0, ef1, wf, mf,
                  accum, semi0, semi1, semg, semo):
    cid = lax.axis_index("c")
    sid = lax.axis_index("s")
    wid = cid * NS + sid
    edge_base = wid * EPW

    zero16 = jnp.zeros((16,), jnp.float32)
    srcs = (srcv0, srcv1)
    dsts = (dstv0, dstv1)
    kvfs = (kvf0, kvf1)
    qfs = (qf0, qf1)
    efs = (ef0, ef1)
    semis = (semi0, semi1)

    def fire_in(ch, b):
        base = edge_base + ch * CHUNK
        pltpu.async_copy(ei_hbm.at[0, pl.ds(base, CHUNK)], srcs[b], semis[b])
        pltpu.async_copy(ei_hbm.at[1, pl.ds(base, CHUNK)], dsts[b], semis[b])
        pltpu.async_copy(e_hbm.at[pl.ds(base, CHUNK)], efs[b], semis[b])

    def wait_in(b):
        pltpu.make_async_copy(ei_hbm.at[0, pl.ds(0, CHUNK)], srcs[b], semis[b]).wait()
        pltpu.make_async_copy(ei_hbm.at[1, pl.ds(0, CHUNK)], dsts[b], semis[b]).wait()
        pltpu.make_async_copy(e_hbm.at[pl.ds(0, CHUNK)], efs[b], semis[b]).wait()

    def fire_gather(b):
        pltpu.async_copy(kvt.at[srcs[b]], kvfs[b], semg)
        pltpu.async_copy(qt.at[dsts[b]], qfs[b], semg)

    def wait_gather(b):
        pltpu.make_async_copy(kvt.at[srcs[b]], kvfs[b], semg).wait()
        pltpu.make_async_copy(qt.at[dsts[b]], qfs[b], semg).wait()

    fire_in(0, 0)
    fire_in(1, 1)

    def zb(i, _):
        mf[i // (ACC_W // 16), pl.ds((i % (ACC_W // 16)) * 16, 16)] = zero16
        return 0
    lax.fori_loop(0, CHUNK * (ACC_W // 16), zb, 0)

    row0 = sid * ROWS_PER_TILE
    nfull = ROWS_PER_TILE // CHUNK

    def zc(i, _):
        pltpu.sync_copy(mf, accum.at[pl.ds(row0 + i * CHUNK, CHUNK)])
        return 0
    lax.fori_loop(0, nfull, zc, 0)
    assert ROWS_PER_TILE == nfull * CHUNK

    plsc.subcore_barrier()

    iota16 = lax.iota(jnp.int32, 16)
    lo8 = iota16 < 8
    and7 = jnp.bitwise_and(iota16, 7)

    def edge_compute(e, kvfb, qfb, efb):
        pacc = zero16
        for j in range(NUM_HEADS // 2):
            slp = pl.ds(32 * j, 32)
            slv = pl.ds(HID + 32 * j, 32)
            h0 = pl.ds(32 * j, OUT_DIM)
            h1 = pl.ds(32 * j + OUT_DIM, OUT_DIM)
            k0, k1 = plsc.unpack(kvfb[e, slp], format=plsc.PackFormat.INTERLEAVED,
                                 preferred_element_type=jnp.float32)
            q0, q1 = plsc.unpack(qfb[e, slp], format=plsc.PackFormat.INTERLEAVED,
                                 preferred_element_type=jnp.float32)
            v0, v1 = plsc.unpack(kvfb[e, slv], format=plsc.PackFormat.INTERLEAVED,
                                 preferred_element_type=jnp.float32)
            e0, e1 = plsc.unpack(efb[e, slp], format=plsc.PackFormat.INTERLEAVED,
                                 preferred_element_type=jnp.float32)
            et0 = k0 * q0 * e0
            et1 = k1 * q1 * e1
            wf[e, h0] = et0
            wf[e, h1] = et1
            s0 = jnp.clip(jnp.sum(et0) * 0.25, -CLAMP, CLAMP)
            s1 = jnp.clip(jnp.sum(et1) * 0.25, -CLAMP, CLAMP)
            p0b = jnp.exp(jnp.broadcast_to(s0, (16,)))
            p1b = jnp.exp(jnp.broadcast_to(s1, (16,)))
            mf[e, h0] = p0b * (v0 + et0)
            mf[e, h1] = p1b * (v1 + et1)
            pacc = jnp.where(iota16 == 2 * j, p0b, pacc)
            pacc = jnp.where(iota16 == 2 * j + 1, p1b, pacc)
        mf[e, pl.ds(HID, 16)] = pacc

    wait_in(0)
    fire_gather(0)

    def loop_body(i, _):
        for b in (0, 1):
            ch = 2 * i + b

            @pl.when(ch + 1 < NCHUNK)
            def _():
                wait_in(1 - b)
                fire_gather(1 - b)

            wait_gather(b)

            @pl.when(ch > 0)
            def _():
                pltpu.make_async_copy(wf, we_hbm.at[pl.ds(0, CHUNK)], semo).wait()

            @plsc.parallel_loop(0, CHUNK, 1, unroll=4)
            def _(e):
                edge_compute(e, kvfs[b], qfs[b], efs[b])

            base = edge_base + ch * CHUNK
            pltpu.async_copy(wf, we_hbm.at[pl.ds(base, CHUNK)], semo)
            pltpu.sync_copy(mf, accum.at[dsts[b]], add=True)

            @pl.when(ch + 2 < NCHUNK)
            def _():
                fire_in(ch + 2, b)
        return 0

    lax.fori_loop(0, NCHUNK // 2, loop_body, 0)
    pltpu.make_async_copy(wf, we_hbm.at[pl.ds(0, CHUNK)], semo).wait()

    plsc.subcore_barrier()

    pltpu.sync_copy(accum.at[pl.ds(row0, ROWS_PER_TILE)],
                    part_hbm.at[cid, pl.ds(row0, ROWS_PER_TILE)])


def _sc_edge_pass(kvt, qt, e, edge_index):
    mesh = plsc.VectorSubcoreMesh(core_axis_name="c", subcore_axis_name="s")
    f = pl.kernel(
        _sc_edge_body,
        out_type=[
            jax.ShapeDtypeStruct((N_EDGES, HID), jnp.float32),
            jax.ShapeDtypeStruct((NC, N_ACC, ACC_W), jnp.float32),
        ],
        mesh=mesh,
        scratch_types=[
            pltpu.VMEM((CHUNK,), jnp.int32),
            pltpu.VMEM((CHUNK,), jnp.int32),
            pltpu.VMEM((CHUNK,), jnp.int32),
            pltpu.VMEM((CHUNK,), jnp.int32),
            pltpu.VMEM((CHUNK, 2 * HID), jnp.bfloat16),
            pltpu.VMEM((CHUNK, 2 * HID), jnp.bfloat16),
            pltpu.VMEM((CHUNK, HID), jnp.bfloat16),
            pltpu.VMEM((CHUNK, HID), jnp.bfloat16),
            pltpu.VMEM((CHUNK, HID), jnp.bfloat16),
            pltpu.VMEM((CHUNK, HID), jnp.bfloat16),
            pltpu.VMEM((CHUNK, HID), jnp.float32),
            pltpu.VMEM((CHUNK, ACC_W), jnp.float32),
            pltpu.VMEM_SHARED((N_ACC, ACC_W), jnp.float32),
            pltpu.SemaphoreType.DMA,
            pltpu.SemaphoreType.DMA,
            pltpu.SemaphoreType.DMA,
            pltpu.SemaphoreType.DMA,
        ],
        compiler_params=pltpu.CompilerParams(
            use_tc_tiling_on_sc=False, needs_layout_passes=False),
    )
    return f(kvt, qt, e, edge_index)


@jax.jit
def kernel(x, edge_index, edge_attr, W_Q, b_Q, W_K, W_E1, b_E1, W_V):
    perm = _interleave_perm()
    nb = 2000
    qt, kvt = pl.pallas_call(
        _proj_nodes_body,
        grid=(N_NODES // nb,),
        in_specs=[
            pl.BlockSpec((nb, IN_DIM), lambda i: (i, 0)),
            pl.BlockSpec((IN_DIM, HID), lambda i: (0, 0)),
            pl.BlockSpec((1, HID), lambda i: (0, 0)),
            pl.BlockSpec((IN_DIM, HID), lambda i: (0, 0)),
            pl.BlockSpec((IN_DIM, HID), lambda i: (0, 0)),
        ],
        out_specs=[
            pl.BlockSpec((nb, HID), lambda i: (i, 0)),
            pl.BlockSpec((nb, 2 * HID), lambda i: (i, 0)),
        ],
        out_shape=[
            jax.ShapeDtypeStruct((N_NODES, HID), jnp.bfloat16),
            jax.ShapeDtypeStruct((N_NODES, 2 * HID), jnp.bfloat16),
        ],
    )(x, W_Q[perm].T, b_Q[perm].reshape(1, HID), W_K[perm].T, W_V[perm].T)

    eb = 2000
    e = pl.pallas_call(
        _proj_edges_body,
        grid=(N_EDGES // eb,),
        in_specs=[
            pl.BlockSpec((eb, IN_DIM), lambda i: (i, 0)),
            pl.BlockSpec((IN_DIM, HID), lambda i: (0, 0)),
            pl.BlockSpec((1, HID), lambda i: (0, 0)),
        ],
        out_specs=pl.BlockSpec((eb, HID), lambda i: (i, 0)),
        out_shape=jax.ShapeDtypeStruct((N_EDGES, HID), jnp.bfloat16),
    )(edge_attr, W_E1[perm].T, b_E1[perm].reshape(1, HID))

    wE, part = _sc_edge_pass(kvt, qt, e, edge_index)

    fb = 1000
    wV = pl.pallas_call(
        _finalize_body,
        grid=(N_NODES // fb,),
        in_specs=[pl.BlockSpec((NC, fb, ACC_W), lambda i: (0, i, 0))],
        out_specs=pl.BlockSpec((fb, HID), lambda i: (i, 0)),
        out_shape=jax.ShapeDtypeStruct((N_NODES, HID), jnp.float32),
    )(part)

    return wV.reshape(N_NODES, NUM_HEADS, OUT_DIM), wE


# speedup vs baseline: 1.3799x; 1.3799x over previous
import functools

import jax
import jax.numpy as jnp
from jax import lax
from jax.experimental import pallas as pl
from jax.experimental.pallas import tpu as pltpu
from jax.experimental.pallas import tpu_sc as plsc

IN_DIM = 128
NUM_HEADS = 8
OUT_DIM = 16
HID = NUM_HEADS * OUT_DIM
N_NODES = 10000
N_EDGES = 320000
CLAMP = 5.0

NC = 2
NS = 16
NW = NC * NS
EPW = N_EDGES // NW
CHUNK = 40
NCHUNK = EPW // CHUNK
ACC_W = 144
N_ACC = 10240
ROWS_PER_TILE = N_ACC // NS


def _interleave_perm():
    p = []
    for j in range(NUM_HEADS // 2):
        for d in range(OUT_DIM):
            p.append(32 * j + d)
            p.append(32 * j + OUT_DIM + d)
    return jnp.array(p, dtype=jnp.int32)


def _proj_nodes_body(x_ref, wq_ref, bq_ref, wk_ref, wv_ref, q_ref, kv_ref):
    xb = x_ref[...]
    q = jnp.dot(xb, wq_ref[...], preferred_element_type=jnp.float32) + bq_ref[...]
    q_ref[...] = q.astype(jnp.bfloat16)
    kv_ref[:, :HID] = jnp.dot(
        xb, wk_ref[...], preferred_element_type=jnp.float32).astype(jnp.bfloat16)
    kv_ref[:, HID:] = jnp.dot(
        xb, wv_ref[...], preferred_element_type=jnp.float32).astype(jnp.bfloat16)


def _proj_edges_body(ea_ref, we1_ref, be1_ref, e_ref):
    e_ref[...] = (
        jnp.dot(ea_ref[...], we1_ref[...], preferred_element_type=jnp.float32)
        + be1_ref[...]
    )


def _finalize_body(part_ref, out_ref):
    s = part_ref[0] + part_ref[1]
    pieces = []
    for h in range(NUM_HEADS):
        num = s[:, h * OUT_DIM:(h + 1) * OUT_DIM]
        den = s[:, HID + h:HID + h + 1] + 1e-16
        pieces.append(num / den)
    out_ref[...] = jnp.concatenate(pieces, axis=1)


def _lane_perm(x, idx):
    dn = lax.GatherDimensionNumbers(
        offset_dims=(), collapsed_slice_dims=(0,), start_index_map=(0,))
    return lax.gather(x, idx[:, None], dn, slice_sizes=(1,),
                      mode=lax.GatherScatterMode.PROMISE_IN_BOUNDS)


def _sc_edge_body(kvt, qt, e_hbm, ei_hbm,
                  we_hbm, part_hbm,
                  srcv0, srcv1, dstv0, dstv1, kvf0, kvf1, qf0, qf1,
                  ef0, ef1, wf, mf,
                  accum, semi0, semi1, semg, semo):
    cid = lax.axis_index("c")
    sid = lax.axis_index("s")
    wid = cid * NS + sid
    edge_base = wid * EPW

    zero16 = jnp.zeros((16,), jnp.float32)
    srcs = (srcv0, srcv1)
    dsts = (dstv0, dstv1)
    kvfs = (kvf0, kvf1)
    qfs = (qf0, qf1)
    efs = (ef0, ef1)
    semis = (semi0, semi1)

    def fire_in(ch, b):
        base = edge_base + ch * CHUNK
        pltpu.async_copy(ei_hbm.at[0, pl.ds(base, CHUNK)], srcs[b], semis[b])
        pltpu.async_copy(ei_hbm.at[1, pl.ds(base, CHUNK)], dsts[b], semis[b])
        pltpu.async_copy(e_hbm.at[pl.ds(base, CHUNK)], efs[b], semis[b])

    def wait_in(b):
        pltpu.make_async_copy(ei_hbm.at[0, pl.ds(0, CHUNK)], srcs[b], semis[b]).wait()
        pltpu.make_async_copy(ei_hbm.at[1, pl.ds(0, CHUNK)], dsts[b], semis[b]).wait()
        pltpu.make_async_copy(e_hbm.at[pl.ds(0, CHUNK)], efs[b], semis[b]).wait()

    def fire_gather(b):
        pltpu.async_copy(kvt.at[srcs[b]], kvfs[b], semg)
        pltpu.async_copy(qt.at[dsts[b]], qfs[b], semg)

    def wait_gather(b):
        pltpu.make_async_copy(kvt.at[srcs[b]], kvfs[b], semg).wait()
        pltpu.make_async_copy(qt.at[dsts[b]], qfs[b], semg).wait()

    fire_in(0, 0)
    fire_in(1, 1)

    def zb(i, _):
        mf[i // (ACC_W // 16), pl.ds((i % (ACC_W // 16)) * 16, 16)] = zero16
        return 0
    lax.fori_loop(0, CHUNK * (ACC_W // 16), zb, 0)

    row0 = sid * ROWS_PER_TILE
    nfull = ROWS_PER_TILE // CHUNK

    def zc(i, _):
        pltpu.sync_copy(mf, accum.at[pl.ds(row0 + i * CHUNK, CHUNK)])
        return 0
    lax.fori_loop(0, nfull, zc, 0)
    assert ROWS_PER_TILE == nfull * CHUNK

    plsc.subcore_barrier()

    iota16 = lax.iota(jnp.int32, 16)
    lo8 = iota16 < 8
    and7 = jnp.bitwise_and(iota16, 7)

    def edge_compute(e, kvfb, qfb, efb):
        pacc = zero16
        for j in range(NUM_HEADS // 2):
            slp = pl.ds(32 * j, 32)
            slv = pl.ds(HID + 32 * j, 32)
            h0 = pl.ds(32 * j, OUT_DIM)
            h1 = pl.ds(32 * j + OUT_DIM, OUT_DIM)
            k0, k1 = plsc.unpack(kvfb[e, slp], format=plsc.PackFormat.INTERLEAVED,
                                 preferred_element_type=jnp.float32)
            q0, q1 = plsc.unpack(qfb[e, slp], format=plsc.PackFormat.INTERLEAVED,
                                 preferred_element_type=jnp.float32)
            v0, v1 = plsc.unpack(kvfb[e, slv], format=plsc.PackFormat.INTERLEAVED,
                                 preferred_element_type=jnp.float32)
            et0 = k0 * q0 * efb[e, h0]
            et1 = k1 * q1 * efb[e, h1]
            wf[e, h0] = et0
            wf[e, h1] = et1
            s0 = jnp.clip(jnp.sum(et0) * 0.25, -CLAMP, CLAMP)
            s1 = jnp.clip(jnp.sum(et1) * 0.25, -CLAMP, CLAMP)
            p0b = jnp.exp(jnp.broadcast_to(s0, (16,)))
            p1b = jnp.exp(jnp.broadcast_to(s1, (16,)))
            mf[e, h0] = p0b * (v0 + et0)
            mf[e, h1] = p1b * (v1 + et1)
            pacc = jnp.where(iota16 == 2 * j, p0b, pacc)
            pacc = jnp.where(iota16 == 2 * j + 1, p1b, pacc)
        mf[e, pl.ds(HID, 16)] = pacc

    wait_in(0)
    fire_gather(0)

    def loop_body(i, _):
        for b in (0, 1):
            ch = 2 * i + b

            @pl.when(ch + 1 < NCHUNK)
            def _():
                wait_in(1 - b)
                fire_gather(1 - b)

            wait_gather(b)

            @pl.when(ch > 0)
            def _():
                pltpu.make_async_copy(wf, we_hbm.at[pl.ds(0, CHUNK)], semo).wait()

            @plsc.parallel_loop(0, CHUNK, 1, unroll=4)
            def _(e):
                edge_compute(e, kvfs[b], qfs[b], efs[b])

            base = edge_base + ch * CHUNK
            pltpu.async_copy(wf, we_hbm.at[pl.ds(base, CHUNK)], semo)
            pltpu.sync_copy(mf, accum.at[dsts[b]], add=True)

            @pl.when(ch + 2 < NCHUNK)
            def _():
                fire_in(ch + 2, b)
        return 0

    lax.fori_loop(0, NCHUNK // 2, loop_body, 0)
    pltpu.make_async_copy(wf, we_hbm.at[pl.ds(0, CHUNK)], semo).wait()

    plsc.subcore_barrier()

    pltpu.sync_copy(accum.at[pl.ds(row0, ROWS_PER_TILE)],
                    part_hbm.at[cid, pl.ds(row0, ROWS_PER_TILE)])


def _sc_edge_pass(kvt, qt, e, edge_index):
    mesh = plsc.VectorSubcoreMesh(core_axis_name="c", subcore_axis_name="s")
    f = pl.kernel(
        _sc_edge_body,
        out_type=[
            jax.ShapeDtypeStruct((N_EDGES, HID), jnp.float32),
            jax.ShapeDtypeStruct((NC, N_ACC, ACC_W), jnp.float32),
        ],
        mesh=mesh,
        scratch_types=[
            pltpu.VMEM((CHUNK,), jnp.int32),
            pltpu.VMEM((CHUNK,), jnp.int32),
            pltpu.VMEM((CHUNK,), jnp.int32),
            pltpu.VMEM((CHUNK,), jnp.int32),
            pltpu.VMEM((CHUNK, 2 * HID), jnp.bfloat16),
            pltpu.VMEM((CHUNK, 2 * HID), jnp.bfloat16),
            pltpu.VMEM((CHUNK, HID), jnp.bfloat16),
            pltpu.VMEM((CHUNK, HID), jnp.bfloat16),
            pltpu.VMEM((CHUNK, HID), jnp.float32),
            pltpu.VMEM((CHUNK, HID), jnp.float32),
            pltpu.VMEM((CHUNK, HID), jnp.float32),
            pltpu.VMEM((CHUNK, ACC_W), jnp.float32),
            pltpu.VMEM_SHARED((N_ACC, ACC_W), jnp.float32),
            pltpu.SemaphoreType.DMA,
            pltpu.SemaphoreType.DMA,
            pltpu.SemaphoreType.DMA,
            pltpu.SemaphoreType.DMA,
        ],
        compiler_params=pltpu.CompilerParams(
            use_tc_tiling_on_sc=False, needs_layout_passes=False),
    )
    return f(kvt, qt, e, edge_index)


@jax.jit
def kernel(x, edge_index, edge_attr, W_Q, b_Q, W_K, W_E1, b_E1, W_V):
    perm = _interleave_perm()
    nb = 2000
    qt, kvt = pl.pallas_call(
        _proj_nodes_body,
        grid=(N_NODES // nb,),
        in_specs=[
            pl.BlockSpec((nb, IN_DIM), lambda i: (i, 0)),
            pl.BlockSpec((IN_DIM, HID), lambda i: (0, 0)),
            pl.BlockSpec((1, HID), lambda i: (0, 0)),
            pl.BlockSpec((IN_DIM, HID), lambda i: (0, 0)),
            pl.BlockSpec((IN_DIM, HID), lambda i: (0, 0)),
        ],
        out_specs=[
            pl.BlockSpec((nb, HID), lambda i: (i, 0)),
            pl.BlockSpec((nb, 2 * HID), lambda i: (i, 0)),
        ],
        out_shape=[
            jax.ShapeDtypeStruct((N_NODES, HID), jnp.bfloat16),
            jax.ShapeDtypeStruct((N_NODES, 2 * HID), jnp.bfloat16),
        ],
    )(x, W_Q[perm].T, b_Q[perm].reshape(1, HID), W_K[perm].T, W_V[perm].T)

    eb = 2000
    e = pl.pallas_call(
        _proj_edges_body,
        grid=(N_EDGES // eb,),
        in_specs=[
            pl.BlockSpec((eb, IN_DIM), lambda i: (i, 0)),
            pl.BlockSpec((IN_DIM, HID), lambda i: (0, 0)),
            pl.BlockSpec((1, HID), lambda i: (0, 0)),
        ],
        out_specs=pl.BlockSpec((eb, HID), lambda i: (i, 0)),
        out_shape=jax.ShapeDtypeStruct((N_EDGES, HID), jnp.float32),
    )(edge_attr, W_E1.T, b_E1.reshape(1, HID))

    wE, part = _sc_edge_pass(kvt, qt, e, edge_index)

    fb = 1000
    wV = pl.pallas_call(
        _finalize_body,
        grid=(N_NODES // fb,),
        in_specs=[pl.BlockSpec((NC, fb, ACC_W), lambda i: (0, i, 0))],
        out_specs=pl.BlockSpec((fb, HID), lambda i: (i, 0)),
        out_shape=jax.ShapeDtypeStruct((N_NODES, HID), jnp.float32),
    )(part)

    return wV.reshape(N_NODES, NUM_HEADS, OUT_DIM), wE
